# Initial kernel scaffold; baseline (speedup 1.0000x reference)
import jax
import jax.numpy as jnp
from jax.experimental import pallas as pl


def kernel(x, edge_index, W_conv, bn_scale, bn_bias, W_lin, b_lin, W_fc_emb, b_fc_emb, W_lin_emb, b_lin_emb, W_fc_reg, b_fc_reg, W_lin_reg, b_lin_reg):
    raise NotImplementedError("write your pallas kernel here")


# speedup vs baseline: 3.0099x; 3.0099x over previous
import functools

import jax
import jax.numpy as jnp
from jax import lax
from jax.experimental import pallas as pl
from jax.experimental.pallas import tpu as pltpu
from jax.experimental.pallas import tpu_sc as plsc

N = 10000
E = 320000
D = 128
NC = 2
NS = 16
NW = NC * NS
CHUNK = 128
T_PER_W = 80
E_PAD = NW * T_PER_W * CHUNK
ACC_ROWS = 10240
ROWS_PER_SUB = ACC_ROWS // NS


def _sc_segment_sum(x, src_p, dst_p):
    mesh = plsc.VectorSubcoreMesh(
        core_axis_name="c", subcore_axis_name="s", num_cores=NC, num_subcores=NS
    )

    @functools.partial(
        pl.kernel,
        out_type=jax.ShapeDtypeStruct((NC, ACC_ROWS, D), jnp.float32),
        mesh=mesh,
        scratch_types=[
            pltpu.VMEM((CHUNK,), jnp.int32),
            pltpu.VMEM((CHUNK,), jnp.int32),
            pltpu.VMEM((CHUNK, D), jnp.float32),
            pltpu.VMEM((8, D), jnp.float32),
            pltpu.VMEM_SHARED((ACC_ROWS, D), jnp.float32),
            pltpu.SemaphoreType.DMA,
        ],
    )
    def k(x_hbm, src_hbm, dst_hbm, out_hbm, src_v, dst_v, rows_v, zbuf, acc, sem):
        c = lax.axis_index("c")
        s = lax.axis_index("s")
        wid = s * NC + c

        zero = jnp.zeros((16,), jnp.float32)
        for r in range(8):
            for q in range(D // 16):
                zbuf[r, pl.ds(q * 16, 16)] = zero

        @pl.loop(0, ROWS_PER_SUB // 8)
        def _zero(i):
            pltpu.sync_copy(zbuf, acc.at[pl.ds(s * ROWS_PER_SUB + i * 8, 8)])

        plsc.subcore_barrier()

        base = wid * (T_PER_W * CHUNK)

        @pl.loop(0, T_PER_W)
        def _edges(t):
            off = base + t * CHUNK
            pltpu.sync_copy(src_hbm.at[pl.ds(off, CHUNK)], src_v)
            pltpu.sync_copy(dst_hbm.at[pl.ds(off, CHUNK)], dst_v)
            pltpu.async_copy(x_hbm.at[src_v], rows_v, sem).wait()
            pltpu.sync_copy(rows_v, acc.at[dst_v], add=True)

        plsc.subcore_barrier()

        pltpu.sync_copy(
            acc.at[pl.ds(s * ROWS_PER_SUB, ROWS_PER_SUB)],
            out_hbm.at[c, pl.ds(s * ROWS_PER_SUB, ROWS_PER_SUB)],
        )

    return k(x, src_p, dst_p)


def _tc_head(p, x, W_conv, bn_scale, bn_bias, W_lin, b_lin,
             W_fc_emb, b_fc_emb, W_lin_emb, b_lin_emb,
             W_fc_reg, b_fc_reg, W_lin_reg, b_lin_reg):
    def body(p_ref, x_ref, wc, g, b, wl, bl, wfe, bfe, wle, ble, wfr, bfr,
             wlr, blr, y_ref, emb_ref, off_ref):
        s = p_ref[0] + p_ref[1] + x_ref[...]
        agg = jnp.dot(s, wc[...], preferred_element_type=jnp.float32)
        mean = jnp.mean(agg, axis=0, keepdims=True)
        cent = agg - mean
        var = jnp.mean(cent * cent, axis=0, keepdims=True)
        inv = lax.rsqrt(var + 1e-4)
        feat = jnp.maximum(cent * inv * g[...] + b[...], 0.0)
        y_ref[...] = jnp.dot(feat, wl[...], preferred_element_type=jnp.float32) + bl[...]
        te = jnp.dot(feat, wfe[...], preferred_element_type=jnp.float32) + bfe[...]
        emb_ref[...] = jnp.dot(te, wle[...], preferred_element_type=jnp.float32) + ble[...]
        tr = jnp.dot(feat, wfr[...], preferred_element_type=jnp.float32) + bfr[...]
        off_ref[...] = jax.nn.sigmoid(
            jnp.dot(tr, wlr[...], preferred_element_type=jnp.float32) + blr[...]
        )

    return pl.pallas_call(
        body,
        out_shape=(
            jax.ShapeDtypeStruct((N, W_lin.shape[1]), jnp.float32),
            jax.ShapeDtypeStruct((N, W_lin_emb.shape[1]), jnp.float32),
            jax.ShapeDtypeStruct((N, W_lin_reg.shape[1]), jnp.float32),
        ),
    )(p, x, W_conv, bn_scale.reshape(1, -1), bn_bias.reshape(1, -1),
      W_lin, b_lin.reshape(1, -1), W_fc_emb, b_fc_emb.reshape(1, -1),
      W_lin_emb, b_lin_emb.reshape(1, -1), W_fc_reg, b_fc_reg.reshape(1, -1),
      W_lin_reg, b_lin_reg.reshape(1, -1))


def kernel(x, edge_index, W_conv, bn_scale, bn_bias, W_lin, b_lin,
           W_fc_emb, b_fc_emb, W_lin_emb, b_lin_emb,
           W_fc_reg, b_fc_reg, W_lin_reg, b_lin_reg):
    src = edge_index[0].astype(jnp.int32)
    dst = edge_index[1].astype(jnp.int32)
    pad = E_PAD - E
    src_p = jnp.concatenate([src, jnp.zeros((pad,), jnp.int32)])
    dst_p = jnp.concatenate([dst, jnp.full((pad,), N, jnp.int32)])
    partials = _sc_segment_sum(x, src_p, dst_p)[:, :N, :]
    return _tc_head(partials, x, W_conv, bn_scale, bn_bias, W_lin, b_lin,
                    W_fc_emb, b_fc_emb, W_lin_emb, b_lin_emb,
                    W_fc_reg, b_fc_reg, W_lin_reg, b_lin_reg)


# speedup vs baseline: 3.7008x; 1.2296x over previous
import functools

import jax
import jax.numpy as jnp
from jax import lax
from jax.experimental import pallas as pl
from jax.experimental.pallas import tpu as pltpu
from jax.experimental.pallas import tpu_sc as plsc

N = 10000
E = 320000
D = 128
NC = 2
NS = 16
NW = NC * NS
CHUNK = 128
T_PER_W = 80
E_PAD = NW * T_PER_W * CHUNK
ACC_ROWS = 10112
ROWS_PER_SUB = ACC_ROWS // NS


NB = 2
PHASES = 2
T_PH = T_PER_W // PHASES


def _sc_segment_sum(x, src2d, dst2d):
    mesh = plsc.VectorSubcoreMesh(
        core_axis_name="c", subcore_axis_name="s", num_cores=NC, num_subcores=NS
    )

    @functools.partial(
        pl.kernel,
        out_type=jax.ShapeDtypeStruct((NC, ACC_ROWS, D), jnp.float32),
        mesh=mesh,
        scratch_types=[
            pltpu.VMEM((T_PH, CHUNK), jnp.int32),
            pltpu.VMEM((T_PH, CHUNK), jnp.int32),
            [pltpu.VMEM((CHUNK, D), jnp.float32) for _ in range(NB)],
            pltpu.VMEM_SHARED((ACC_ROWS, D), jnp.float32),
            pltpu.SemaphoreType.DMA,
            [pltpu.SemaphoreType.DMA for _ in range(NB)],
            [pltpu.SemaphoreType.DMA for _ in range(NB)],
        ],
    )
    def k(x_hbm, src_hbm, dst_hbm, out_hbm, src_slab, dst_slab, rows,
          acc, isem, gsem, ssem):
        c = lax.axis_index("c")
        s = lax.axis_index("s")
        wid = s * NC + c

        zero = jnp.zeros((16,), jnp.float32)

        @pl.loop(0, CHUNK)
        def _zrow(r):
            for q in range(D // 16):
                rows[0][r, pl.ds(q * 16, 16)] = zero

        for i in range(4):
            pltpu.sync_copy(rows[0], acc.at[pl.ds(s * ROWS_PER_SUB + i * CHUNK, CHUNK)])
        pltpu.sync_copy(rows[0].at[pl.ds(0, ROWS_PER_SUB - 4 * CHUNK)],
                        acc.at[pl.ds(s * ROWS_PER_SUB + 4 * CHUNK,
                                     ROWS_PER_SUB - 4 * CHUNK)])

        def start_gather(t, b):
            pltpu.async_copy(x_hbm.at[src_slab.at[t]], rows[b], gsem[b])

        def wait_gather(b):
            pltpu.make_async_copy(x_hbm.at[src_slab.at[0]], rows[b], gsem[b]).wait()

        def start_scatter(t, b):
            pltpu.async_copy(rows[b], acc.at[dst_slab.at[t]], ssem[b], add=True)

        def wait_scatter(b):
            pltpu.make_async_copy(rows[b], acc.at[dst_slab.at[0]], ssem[b]).wait()

        plsc.subcore_barrier()

        for ph in range(PHASES):
            slab_base = wid * T_PER_W + ph * T_PH
            pltpu.async_copy(src_hbm.at[pl.ds(slab_base, T_PH)], src_slab, isem)
            pltpu.async_copy(dst_hbm.at[pl.ds(slab_base, T_PH)], dst_slab, isem)
            pltpu.make_async_copy(src_hbm.at[pl.ds(0, T_PH)], src_slab, isem).wait()
            pltpu.make_async_copy(dst_hbm.at[pl.ds(0, T_PH)], dst_slab, isem).wait()

            for b in range(NB):
                start_gather(b, b)

            @pl.loop(0, (T_PH - NB) // NB)
            def _edges(g):
                for b in range(NB):
                    t = g * NB + b
                    wait_gather(b)
                    start_scatter(t, b)
                    wait_scatter(b)
                    start_gather(t + NB, b)

            for b in range(NB):
                t = T_PH - NB + b
                wait_gather(b)
                start_scatter(t, b)
                wait_scatter(b)

        plsc.subcore_barrier()

        pltpu.sync_copy(
            acc.at[pl.ds(s * ROWS_PER_SUB, ROWS_PER_SUB)],
            out_hbm.at[c, pl.ds(s * ROWS_PER_SUB, ROWS_PER_SUB)],
        )

    return k(x, src2d, dst2d)


def _tc_head(p, x, W_conv, bn_scale, bn_bias, W_lin, b_lin,
             W_fc_emb, b_fc_emb, W_lin_emb, b_lin_emb,
             W_fc_reg, b_fc_reg, W_lin_reg, b_lin_reg):
    def body(p_ref, x_ref, wc, g, b, wl, bl, wfe, bfe, wle, ble, wfr, bfr,
             wlr, blr, y_ref, emb_ref, off_ref):
        s = p_ref[0] + p_ref[1] + x_ref[...]
        agg = jnp.dot(s, wc[...], preferred_element_type=jnp.float32)
        mean = jnp.mean(agg, axis=0, keepdims=True)
        cent = agg - mean
        var = jnp.mean(cent * cent, axis=0, keepdims=True)
        inv = lax.rsqrt(var + 1e-4)
        feat = jnp.maximum(cent * inv * g[...] + b[...], 0.0)
        y_ref[...] = jnp.dot(feat, wl[...], preferred_element_type=jnp.float32) + bl[...]
        te = jnp.dot(feat, wfe[...], preferred_element_type=jnp.float32) + bfe[...]
        emb_ref[...] = jnp.dot(te, wle[...], preferred_element_type=jnp.float32) + ble[...]
        tr = jnp.dot(feat, wfr[...], preferred_element_type=jnp.float32) + bfr[...]
        off_ref[...] = jax.nn.sigmoid(
            jnp.dot(tr, wlr[...], preferred_element_type=jnp.float32) + blr[...]
        )

    return pl.pallas_call(
        body,
        out_shape=(
            jax.ShapeDtypeStruct((N, W_lin.shape[1]), jnp.float32),
            jax.ShapeDtypeStruct((N, W_lin_emb.shape[1]), jnp.float32),
            jax.ShapeDtypeStruct((N, W_lin_reg.shape[1]), jnp.float32),
        ),
    )(p, x, W_conv, bn_scale.reshape(1, -1), bn_bias.reshape(1, -1),
      W_lin, b_lin.reshape(1, -1), W_fc_emb, b_fc_emb.reshape(1, -1),
      W_lin_emb, b_lin_emb.reshape(1, -1), W_fc_reg, b_fc_reg.reshape(1, -1),
      W_lin_reg, b_lin_reg.reshape(1, -1))


def kernel(x, edge_index, W_conv, bn_scale, bn_bias, W_lin, b_lin,
           W_fc_emb, b_fc_emb, W_lin_emb, b_lin_emb,
           W_fc_reg, b_fc_reg, W_lin_reg, b_lin_reg):
    src = edge_index[0].astype(jnp.int32)
    dst = edge_index[1].astype(jnp.int32)
    pad = E_PAD - E
    src_p = jnp.concatenate([src, jnp.zeros((pad,), jnp.int32)]).reshape(-1, CHUNK)
    dst_p = jnp.concatenate([dst, jnp.full((pad,), N, jnp.int32)]).reshape(-1, CHUNK)
    partials = _sc_segment_sum(x, src_p, dst_p)[:, :N, :]
    return _tc_head(partials, x, W_conv, bn_scale, bn_bias, W_lin, b_lin,
                    W_fc_emb, b_fc_emb, W_lin_emb, b_lin_emb,
                    W_fc_reg, b_fc_reg, W_lin_reg, b_lin_reg)
